# Initial kernel scaffold; baseline (speedup 1.0000x reference)
import jax
import jax.numpy as jnp
from jax.experimental import pallas as pl


def kernel(input_batch, router_w, w1, b1, w2, b2):
    raise NotImplementedError("write your pallas kernel here")


# speedup vs baseline: 1.8664x; 1.8664x over previous
import jax
import jax.numpy as jnp
from jax.experimental import pallas as pl
from jax.experimental.pallas import tpu as pltpu

E = 8
K = 2
D = 1024
F = 2048
N = 2048
EP = 128
T = 512


def _router_kernel(x_ref, rw_ref, comb_ref, loss_ref):
    x = x_ref[...]
    rw = rw_ref[...]
    logits = jnp.dot(x, rw, preferred_element_type=jnp.float32)
    col = jax.lax.broadcasted_iota(jnp.int32, (N, EP), 1)
    valid = col < E
    logits = jnp.where(valid, logits, -1e30)
    m = jnp.max(logits, axis=1, keepdims=True)
    ex = jnp.where(valid, jnp.exp(logits - m), 0.0)
    s = jnp.sum(ex, axis=1, keepdims=True)
    probs = ex / s
    p1 = jnp.max(probs, axis=1, keepdims=True)
    i1 = jnp.min(jnp.where(probs == p1, col, EP), axis=1, keepdims=True)
    oh1 = (col == i1).astype(jnp.float32)
    probs2 = jnp.where(col == i1, -1.0, probs)
    p2 = jnp.max(probs2, axis=1, keepdims=True)
    i2 = jnp.min(jnp.where(probs2 == p2, col, EP), axis=1, keepdims=True)
    oh2 = (col == i2).astype(jnp.float32)
    tot = p1 + p2
    comb = oh1 * (p1 / tot) + oh2 * (p2 / tot)
    comb_ref[...] = comb
    me = jnp.sum(probs, axis=0, keepdims=True) / N
    ce = jnp.sum(oh1 + oh2, axis=0, keepdims=True) / (N * K)
    loss_ref[0, 0] = E * jnp.sum(me * ce)


def _ffn_kernel(x_ref, w1_ref, b1_ref, w2_ref, b2_ref, comb_ref,
                out_ref, acc_ref):
    e = pl.program_id(0)
    t = pl.program_id(1)
    x = x_ref[...]
    h = jnp.dot(x, w1_ref[0], preferred_element_type=jnp.float32)
    h = jnp.maximum(h + b1_ref[0], 0.0)
    y = jnp.dot(h, w2_ref[0], preferred_element_type=jnp.float32)
    y = y + b2_ref[0]
    colE = jax.lax.broadcasted_iota(jnp.int32, (T, EP), 1)
    wts = jnp.sum(jnp.where(colE == e, comb_ref[...], 0.0),
                  axis=1, keepdims=True)
    contrib = wts * y
    sl = pl.ds(t * T, T)

    @pl.when(e == 0)
    def _init():
        acc_ref[sl, :] = contrib

    @pl.when(e != 0)
    def _acc():
        acc_ref[sl, :] = acc_ref[sl, :] + contrib

    @pl.when(e == E - 1)
    def _emit():
        out_ref[...] = acc_ref[sl, :]


def kernel(input_batch, router_w, w1, b1, w2, b2):
    rw_pad = jnp.pad(router_w, ((0, 0), (0, EP - E)))
    comb, loss = pl.pallas_call(
        _router_kernel,
        out_shape=[
            jax.ShapeDtypeStruct((N, EP), jnp.float32),
            jax.ShapeDtypeStruct((1, 1), jnp.float32),
        ],
        out_specs=[
            pl.BlockSpec((N, EP), lambda: (0, 0)),
            pl.BlockSpec(memory_space=pltpu.SMEM),
        ],
    )(input_batch, rw_pad)

    b1r = b1.reshape(E, 1, F)
    b2r = b2.reshape(E, 1, D)
    out = pl.pallas_call(
        _ffn_kernel,
        grid=(E, N // T),
        in_specs=[
            pl.BlockSpec((T, D), lambda e, t: (t, 0)),
            pl.BlockSpec((1, D, F), lambda e, t: (e, 0, 0)),
            pl.BlockSpec((1, 1, F), lambda e, t: (e, 0, 0)),
            pl.BlockSpec((1, F, D), lambda e, t: (e, 0, 0)),
            pl.BlockSpec((1, 1, D), lambda e, t: (e, 0, 0)),
            pl.BlockSpec((T, EP), lambda e, t: (t, 0)),
        ],
        out_specs=pl.BlockSpec(
            (T, D), lambda e, t: (jnp.where(e == E - 1, t, 0), 0)),
        out_shape=jax.ShapeDtypeStruct((N, D), jnp.float32),
        scratch_shapes=[pltpu.VMEM((N, D), jnp.float32)],
    )(input_batch, w1, b1r, w2, b2r, comb)

    return out, loss[0, 0]
